# baseline (speedup 1.0000x reference)
import jax
import jax.numpy as jnp
from jax import lax
from jax.experimental import pallas as pl
from jax.experimental.pallas import tpu as pltpu
from jax.experimental.pallas import tpu_sc as plsc

N = 10000
E = 320000
D_IN = 128
H1 = 64
C2 = 16
W1ROW = H1 + 8
W2ROW = 2 * C2
NEG = 0.2

NP = 10240
RANGE = NP // 32
CH = 128
SCCH = 2048
ETOT = 335872
NSCAN = ETOT // SCCH
LCAP = 11520
NCH = LCAP // CH
ROWBLK = 1024
BMUL = 13108
HIGHEST = jax.lax.Precision.HIGHEST


def _tc1_body(x_ref, w1_ref, asrc_ref, adst_ref, xp_ref, as_ref, ad_ref):
    xp = jnp.dot(x_ref[...], w1_ref[...], preferred_element_type=jnp.float32,
                 precision=HIGHEST)
    xp_ref[...] = xp
    as_ref[...] = jnp.dot(xp, asrc_ref[...], preferred_element_type=jnp.float32,
                          precision=HIGHEST)
    ad_ref[...] = jnp.dot(xp, adst_ref[...], preferred_element_type=jnp.float32,
                          precision=HIGHEST)


def _tc1(xpad, W1, Asrc, Adst):
    return pl.pallas_call(
        _tc1_body,
        grid=(NP // ROWBLK,),
        in_specs=[
            pl.BlockSpec((ROWBLK, D_IN), lambda i: (i, 0)),
            pl.BlockSpec((D_IN, H1), lambda i: (0, 0)),
            pl.BlockSpec((H1, 8), lambda i: (0, 0)),
            pl.BlockSpec((H1, 8), lambda i: (0, 0)),
        ],
        out_specs=[
            pl.BlockSpec((ROWBLK, H1), lambda i: (i, 0)),
            pl.BlockSpec((ROWBLK, 8), lambda i: (i, 0)),
            pl.BlockSpec((ROWBLK, 8), lambda i: (i, 0)),
        ],
        out_shape=[
            jax.ShapeDtypeStruct((NP, H1), jnp.float32),
            jax.ShapeDtypeStruct((NP, 8), jnp.float32),
            jax.ShapeDtypeStruct((NP, 8), jnp.float32),
        ],
    )(xpad, W1, Asrc, Adst)


def _tc2_body(acc_ref, b1_ref, w2_ref, bmat_ref, a2s_ref, a2d_ref,
              xp2_ref, as2_ref, ad2_ref):
    acc = acc_ref[...]
    num = acc[:, :H1]
    ssum = acc[:, H1:H1 + 8]
    s64 = jnp.dot(ssum, bmat_ref[...], preferred_element_type=jnp.float32,
                  precision=HIGHEST)
    h = num / (s64 + 1e-16) + b1_ref[...]
    h = jnp.where(h > 0, h, jnp.exp(h) - 1.0)
    xp2 = jnp.dot(h, w2_ref[...], preferred_element_type=jnp.float32,
                  precision=HIGHEST)
    xp2_ref[...] = xp2
    as2_ref[...] = jnp.dot(xp2, a2s_ref[...], preferred_element_type=jnp.float32,
                           precision=HIGHEST)
    ad2_ref[...] = jnp.dot(xp2, a2d_ref[...], preferred_element_type=jnp.float32,
                           precision=HIGHEST)


def _tc2(outp, b1r, W2, Bmat, a2s, a2d):
    return pl.pallas_call(
        _tc2_body,
        grid=(NP // ROWBLK,),
        in_specs=[
            pl.BlockSpec((ROWBLK, W1ROW), lambda i: (i, 0)),
            pl.BlockSpec((1, H1), lambda i: (0, 0)),
            pl.BlockSpec((H1, C2), lambda i: (0, 0)),
            pl.BlockSpec((8, H1), lambda i: (0, 0)),
            pl.BlockSpec((C2, 8), lambda i: (0, 0)),
            pl.BlockSpec((C2, 8), lambda i: (0, 0)),
        ],
        out_specs=[
            pl.BlockSpec((ROWBLK, C2), lambda i: (i, 0)),
            pl.BlockSpec((ROWBLK, 8), lambda i: (i, 0)),
            pl.BlockSpec((ROWBLK, 8), lambda i: (i, 0)),
        ],
        out_shape=[
            jax.ShapeDtypeStruct((NP, C2), jnp.float32),
            jax.ShapeDtypeStruct((NP, 8), jnp.float32),
            jax.ShapeDtypeStruct((NP, 8), jnp.float32),
        ],
    )(outp, b1r, W2, Bmat, a2s, a2d)


def _tc3_body(acc_ref, b2_ref, o_ref):
    acc = acc_ref[...]
    num = acc[:, :C2]
    ssum = acc[:, C2:C2 + 1]
    o_ref[...] = num / (ssum + 1e-16) + b2_ref[...]


def _tc3(out2p, b2r):
    return pl.pallas_call(
        _tc3_body,
        grid=(NP // ROWBLK,),
        in_specs=[
            pl.BlockSpec((ROWBLK, W2ROW), lambda i: (i, 0)),
            pl.BlockSpec((1, C2), lambda i: (0, 0)),
        ],
        out_specs=pl.BlockSpec((ROWBLK, C2), lambda i: (i, 0)),
        out_shape=jax.ShapeDtypeStruct((NP, C2), jnp.float32),
    )(out2p, b2r)


_SC_PARAMS = pltpu.CompilerParams(use_tc_tiling_on_sc=False,
                                  needs_layout_passes=False)


def _sc1_body(src_hbm, dst_hbm, xp_hbm, as_hbm, ad_hbm, m_hbm,
              out_hbm, srcl_hbm, dstl_hbm, cnt_hbm,
              sscan, dscan, srcl, dstl, as_b, xp_b, e_b, ad_own, acc,
              m_v, cv_b, gsem0, gsem1):
    cid = lax.axis_index("c")
    sid = lax.axis_index("s")
    wid = cid * 16 + sid
    base = wid * RANGE
    lane = lax.iota(jnp.int32, 16)
    rowpat = lax.shift_right_logical(lane, 3)
    colpat = lane & 7
    fzero = jnp.zeros((16,), jnp.float32)
    izero = jnp.zeros((16,), jnp.int32)
    gsem = (gsem0, gsem1)

    @pl.loop(0, RANGE)
    def _zacc(r):
        for j in range(4):
            acc[r, pl.ds(j * 16, 16)] = fzero

    @pl.loop(0, RANGE // 2)
    def _zacc_tail(i):
        plsc.store_scatter(acc, [2 * i + rowpat, H1 + colpat], fzero)

    basev = izero + base
    @pl.loop(0, LCAP // 16)
    def _pfill(i):
        srcl[pl.ds(i * 16, 16)] = izero
        dstl[pl.ds(i * 16, 16)] = basev

    pltpu.sync_copy(ad_hbm.at[pl.ds(base, RANGE)], ad_own)
    pltpu.sync_copy(m_hbm, m_v)
    mv = m_v[...]

    def sfire(b, i):
        o = i * SCCH
        pltpu.async_copy(src_hbm.at[pl.ds(o, SCCH)], sscan.at[b], gsem[b])
        pltpu.async_copy(dst_hbm.at[pl.ds(o, SCCH)], dscan.at[b], gsem[b])

    sfire(0, 0)
    sfire(1, 1)

    @pl.loop(0, NSCAN, step=2, init_carry=0)
    def _scan(g, cnt0):
        cnt = cnt0
        for b in range(2):
            gi = g + b
            pltpu.make_async_copy(src_hbm.at[pl.ds(0, SCCH)], sscan.at[b],
                                  gsem[b]).wait()
            pltpu.make_async_copy(dst_hbm.at[pl.ds(0, SCCH)], dscan.at[b],
                                  gsem[b]).wait()

            @pl.loop(0, SCCH // 16, init_carry=cnt, unroll=2)
            def _groups(i, off):
                sv = sscan[b, pl.ds(i * 16, 16)]
                dv = dscan[b, pl.ds(i * 16, 16)]
                bkt = lax.shift_right_logical(dv * BMUL, 22)
                m = jnp.logical_and(bkt == wid, off <= LCAP - 16)
                plsc.store_compressed(srcl.at[pl.ds(off, 16)], sv, mask=m)
                plsc.store_compressed(dstl.at[pl.ds(off, 16)], dv, mask=m)
                return off + jnp.sum(jnp.where(m, 1, 0))

            cnt = _groups

            @pl.when(gi + 2 < NSCAN)
            def _():
                sfire(b, gi + 2)
        return cnt

    cnt = _scan

    cv_b[...] = jnp.where(lane == 0, cnt, 0)
    pltpu.sync_copy(cv_b, cnt_hbm.at[wid])
    pltpu.sync_copy(srcl, srcl_hbm.at[wid])
    pltpu.sync_copy(dstl, dstl_hbm.at[wid])

    def pfire(b, ci):
        o = ci * CH
        pltpu.async_copy(as_hbm.at[srcl.at[pl.ds(o, CH)]], as_b.at[b], gsem[b])
        pltpu.async_copy(xp_hbm.at[srcl.at[pl.ds(o, CH)]], xp_b.at[b], gsem[b])

    @pl.when(cnt > 0)
    def _():
        pfire(0, 0)

    @pl.loop(0, NCH, step=2)
    def _proc(g):
        for b in range(2):
            ci = g + b
            active = ci * CH < cnt
            nxt = (ci + 1) * CH < cnt

            @pl.when(active)
            def _():
                pltpu.make_async_copy(as_hbm.at[srcl.at[pl.ds(0, CH)]],
                                      as_b.at[b], gsem[b]).wait()
                pltpu.make_async_copy(xp_hbm.at[srcl.at[pl.ds(0, CH)]],
                                      xp_b.at[b], gsem[b]).wait()

            @pl.when(nxt)
            def _():
                pfire(1 - b, ci + 1)

            @pl.when(active)
            def _():
                cbase = ci * CH

                @pl.loop(0, 64, unroll=2)
                def _alpha(i):
                    row = 2 * i + rowpat
                    dloc = plsc.load_gather(dstl, [cbase + row]) - base
                    va = plsc.load_gather(as_b.at[b], [row, colpat])
                    vd = plsc.load_gather(ad_own, [dloc, colpat])
                    al = va + vd
                    al = jnp.where(al > 0, al, al * NEG)
                    ev = jnp.exp(al - mv)
                    ev = jnp.where(cbase + row < cnt, ev, 0.0)
                    plsc.store_scatter(e_b, [row, colpat], ev)

                @pl.loop(0, CH, unroll=2)
                def _msg(k):
                    ks = izero + k
                    dk = plsc.load_gather(dstl, [cbase + ks]) - base
                    for j in range(4):
                        evj = plsc.load_gather(e_b, [ks, rowpat + 2 * j])
                        mj = evj * xp_b[b, k, pl.ds(j * 16, 16)]
                        plsc.addupdate_scatter(acc, [dk, j * 16 + lane], mj)
                    em = plsc.load_gather(e_b, [ks, colpat])
                    plsc.addupdate_scatter(acc, [dk, H1 + colpat], em,
                                           mask=lane < 8)

    pltpu.sync_copy(acc, out_hbm.at[pl.ds(base, RANGE)])


def _sc1(src, dst, xp1, as1, ad1, m1v):
    mesh = plsc.VectorSubcoreMesh(core_axis_name="c", subcore_axis_name="s")
    f = pl.kernel(
        _sc1_body,
        out_type=[
            jax.ShapeDtypeStruct((NP, W1ROW), jnp.float32),
            jax.ShapeDtypeStruct((32, LCAP), jnp.int32),
            jax.ShapeDtypeStruct((32, LCAP), jnp.int32),
            jax.ShapeDtypeStruct((32, 16), jnp.int32),
        ],
        mesh=mesh,
        compiler_params=_SC_PARAMS,
        scratch_types=[
            pltpu.VMEM((2, SCCH), jnp.int32),
            pltpu.VMEM((2, SCCH), jnp.int32),
            pltpu.VMEM((LCAP,), jnp.int32),
            pltpu.VMEM((LCAP,), jnp.int32),
            pltpu.VMEM((2, CH, 8), jnp.float32),
            pltpu.VMEM((2, CH, H1), jnp.float32),
            pltpu.VMEM((CH, 8), jnp.float32),
            pltpu.VMEM((RANGE, 8), jnp.float32),
            pltpu.VMEM((RANGE, W1ROW), jnp.float32),
            pltpu.VMEM((16,), jnp.float32),
            pltpu.VMEM((16,), jnp.int32),
            pltpu.SemaphoreType.DMA,
            pltpu.SemaphoreType.DMA,
        ],
    )
    return f(src, dst, xp1, as1, ad1, m1v)


def _sc2_body(srcl_hbm, dstl_hbm, cnt_hbm, xp2_hbm, as2_hbm, ad2_hbm, m2_hbm,
              out2_hbm,
              srcl, dstl, as2_v, ad2_v, xp2_b, e_b2, acc2, m2_v, cv_b,
              gsem0, gsem1):
    cid = lax.axis_index("c")
    sid = lax.axis_index("s")
    wid = cid * 16 + sid
    base = wid * RANGE
    lane = lax.iota(jnp.int32, 16)
    fzero = jnp.zeros((16,), jnp.float32)
    izero = jnp.zeros((16,), jnp.int32)
    col16 = izero + C2
    gsem = (gsem0, gsem1)

    @pl.loop(0, RANGE)
    def _zacc(r):
        acc2[r, pl.ds(0, 16)] = fzero
        acc2[r, pl.ds(16, 16)] = fzero

    pltpu.sync_copy(srcl_hbm.at[wid], srcl)
    pltpu.sync_copy(dstl_hbm.at[wid], dstl)
    pltpu.sync_copy(cnt_hbm.at[wid], cv_b)
    cnt = cv_b[...][0]
    pltpu.sync_copy(as2_hbm, as2_v)
    pltpu.sync_copy(ad2_hbm, ad2_v)
    pltpu.sync_copy(m2_hbm, m2_v)
    mv = m2_v[...]

    def pfire(b, ci):
        pltpu.async_copy(xp2_hbm.at[srcl.at[pl.ds(ci * CH, CH)]],
                         xp2_b.at[b], gsem[b])

    @pl.when(cnt > 0)
    def _():
        pfire(0, 0)

    @pl.loop(0, NCH, step=2)
    def _proc(g):
        for b in range(2):
            ci = g + b
            active = ci * CH < cnt
            nxt = (ci + 1) * CH < cnt

            @pl.when(active)
            def _():
                pltpu.make_async_copy(xp2_hbm.at[srcl.at[pl.ds(0, CH)]],
                                      xp2_b.at[b], gsem[b]).wait()

            @pl.when(nxt)
            def _():
                pfire(1 - b, ci + 1)

            @pl.when(active)
            def _():
                cbase = ci * CH

                @pl.loop(0, 8)
                def _alpha(i):
                    sv = srcl[pl.ds(cbase + i * 16, 16)]
                    dv = dstl[pl.ds(cbase + i * 16, 16)]
                    va = plsc.load_gather(as2_v, [sv])
                    vd = plsc.load_gather(ad2_v, [dv])
                    al = va + vd
                    al = jnp.where(al > 0, al, al * NEG)
                    ev = jnp.exp(al - mv)
                    ev = jnp.where(cbase + i * 16 + lane < cnt, ev, 0.0)
                    e_b2[pl.ds(i * 16, 16)] = ev

                @pl.loop(0, CH, unroll=4)
                def _msg(k):
                    ks = izero + k
                    dk = plsc.load_gather(dstl, [cbase + ks]) - base
                    ev = plsc.load_gather(e_b2, [ks])
                    mj = ev * xp2_b[b, k, pl.ds(0, 16)]
                    plsc.addupdate_scatter(acc2, [dk, lane], mj)
                    plsc.addupdate_scatter(acc2, [dk, col16], ev, mask=lane < 1)

    pltpu.sync_copy(acc2, out2_hbm.at[pl.ds(base, RANGE)])


def _sc2(srcl, dstl, cnts, xp2, as2, ad2, m2v):
    mesh = plsc.VectorSubcoreMesh(core_axis_name="c", subcore_axis_name="s")
    f = pl.kernel(
        _sc2_body,
        out_type=[
            jax.ShapeDtypeStruct((NP, W2ROW), jnp.float32),
        ],
        mesh=mesh,
        compiler_params=_SC_PARAMS,
        scratch_types=[
            pltpu.VMEM((LCAP,), jnp.int32),
            pltpu.VMEM((LCAP,), jnp.int32),
            pltpu.VMEM((NP,), jnp.float32),
            pltpu.VMEM((NP,), jnp.float32),
            pltpu.VMEM((2, CH, C2), jnp.float32),
            pltpu.VMEM((CH,), jnp.float32),
            pltpu.VMEM((RANGE, W2ROW), jnp.float32),
            pltpu.VMEM((16,), jnp.float32),
            pltpu.VMEM((16,), jnp.int32),
            pltpu.SemaphoreType.DMA,
            pltpu.SemaphoreType.DMA,
        ],
    )
    return f(srcl, dstl, cnts, xp2, as2, ad2, m2v)[0]


def kernel(x, edge_index, W1, att_src1, att_dst1, b1, W2, att_src2, att_dst2, b2):
    f32 = jnp.float32
    pad = ETOT - E - N
    loop = jnp.arange(N, dtype=jnp.int32)
    dummy = jnp.full((pad,), N, jnp.int32)
    src = jnp.concatenate([edge_index[0], loop, dummy])
    dst = jnp.concatenate([edge_index[1], loop, dummy])

    xpad = jnp.pad(x, ((0, NP - N), (0, 0)))
    eye8 = jnp.eye(8, dtype=f32)
    Asrc = jnp.einsum("hc,hg->hcg", att_src1[0], eye8).reshape(H1, 8)
    Adst = jnp.einsum("hc,hg->hcg", att_dst1[0], eye8).reshape(H1, 8)

    xp1, as1, ad1 = _tc1(xpad, W1, Asrc, Adst)
    m1 = jax.nn.relu(jnp.max(as1, axis=0) + jnp.max(ad1, axis=0))
    m1v = jnp.tile(m1, 2)

    outp, srcl, dstl, cnts = _sc1(src, dst, xp1, as1, ad1, m1v)

    Bmat = jnp.kron(eye8, jnp.ones((1, 8), f32))
    a2s = jnp.zeros((C2, 8), f32).at[:, 0].set(att_src2.reshape(C2))
    a2d = jnp.zeros((C2, 8), f32).at[:, 0].set(att_dst2.reshape(C2))
    xp2, as2o, ad2o = _tc2(outp, b1.reshape(1, H1), W2, Bmat, a2s, a2d)
    as2 = as2o[:, 0]
    ad2 = ad2o[:, 0]
    m2 = jax.nn.relu(jnp.max(as2) + jnp.max(ad2))
    m2v = jnp.full((16,), m2, f32)

    out2p = _sc2(srcl, dstl, cnts, xp2, as2, ad2, m2v)
    out = _tc3(out2p, b2.reshape(1, C2))
    return out[:N]
